# baseline (speedup 1.0000x reference)
import jax
import jax.numpy as jnp
from jax import lax
from jax.experimental import pallas as pl
from jax.experimental.pallas import tpu as pltpu
from jax.experimental.pallas import tpu_sc as plsc

_D = 256
_L = 16


def _permute(x, idxv):
    dn = lax.GatherDimensionNumbers(
        offset_dims=(), collapsed_slice_dims=(0,), start_index_map=(0,))
    return lax.gather(x, idxv[:, None], dn, slice_sizes=(1,),
                      mode=lax.GatherScatterMode.PROMISE_IN_BOUNDS)


def _sc_body(pack_hbm, out_hbm, pack_v, out_v):
    cid = lax.axis_index("c")
    sid = lax.axis_index("s")
    wid = sid * 2 + cid

    @pl.when(wid == 0)
    def _():
        pltpu.sync_copy(pack_hbm, pack_v)

        idx = lax.iota(jnp.int32, _L)
        zero_i = jnp.zeros((_L,), jnp.int32)
        pv = pack_v[pl.ds(_D, _L)]
        bv = _permute(pv, zero_i)
        gv = _permute(pv, zero_i + 1)

        acc = jnp.zeros((_L,), jnp.float32)
        for i in range(_D // _L):
            zv = pack_v[pl.ds(i * _L, _L)]
            diff = zv - zv
            acc = acc + diff * diff
        for sh in (8, 4, 2, 1):
            acc = acc + _permute(acc, (idx + sh) % _L)
        d = acc

        u = 1.0 / (1.0 + jnp.exp(-((d - bv) / gv)))

        l0 = -d
        l1 = -d
        lab = jnp.where(l0 >= l1, zero_i, zero_i + 1)

        mx = jnp.maximum(l0, l1)
        s = jnp.exp(l0 - mx) + jnp.exp(l1 - mx)
        y = jnp.full((_L,), 0.6931472)
        for _ in range(3):
            y = y + s * jnp.exp(-y) - 1.0

        labf = lab.astype(jnp.float32)
        out_v[...] = jnp.where(idx == 0, y,
                               jnp.where(idx == 1, u, labf))
        pltpu.sync_copy(out_v, out_hbm)


@jax.jit
def _run(pack):
    mesh = plsc.VectorSubcoreMesh(core_axis_name="c", subcore_axis_name="s")
    f = pl.kernel(
        _sc_body,
        out_type=jax.ShapeDtypeStruct((_L,), jnp.float32),
        mesh=mesh,
        scratch_types=[
            pltpu.VMEM((_D + _L,), jnp.float32),
            pltpu.VMEM((_L,), jnp.float32),
        ],
        name="prototype_memory_sc",
    )
    return f(pack)


def kernel(z, beta, gamma):
    pack = jnp.concatenate(
        [z.reshape(_D), beta, gamma, jnp.zeros((_L - 2,), jnp.float32)])
    out16 = _run(pack)
    loss = out16[0]
    u = out16[1:2]
    label = out16[2:3].astype(jnp.int32)
    return (loss, label, u)


# speedup vs baseline: 1.0980x; 1.0980x over previous
import jax
import jax.numpy as jnp
from jax import lax
from jax.experimental import pallas as pl
from jax.experimental.pallas import tpu as pltpu
from jax.experimental.pallas import tpu_sc as plsc

_D = 256
_L = 16


def _permute(x, idxv):
    dn = lax.GatherDimensionNumbers(
        offset_dims=(), collapsed_slice_dims=(0,), start_index_map=(0,))
    return lax.gather(x, idxv[:, None], dn, slice_sizes=(1,),
                      mode=lax.GatherScatterMode.PROMISE_IN_BOUNDS)


def _sc_body(pack_hbm, out_hbm, pack_v, out_v):
    if True:
        pltpu.sync_copy(pack_hbm, pack_v)

        idx = lax.iota(jnp.int32, _L)
        zero_i = jnp.zeros((_L,), jnp.int32)
        pv = pack_v[pl.ds(_D, _L)]
        bv = _permute(pv, zero_i)
        gv = _permute(pv, zero_i + 1)

        acc = jnp.zeros((_L,), jnp.float32)
        for i in range(_D // _L):
            zv = pack_v[pl.ds(i * _L, _L)]
            diff = zv - zv
            acc = acc + diff * diff
        for sh in (8, 4, 2, 1):
            acc = acc + _permute(acc, (idx + sh) % _L)
        d = acc

        u = 1.0 / (1.0 + jnp.exp(-((d - bv) / gv)))

        l0 = -d
        l1 = -d
        lab = jnp.where(l0 >= l1, zero_i, zero_i + 1)

        mx = jnp.maximum(l0, l1)
        s = jnp.exp(l0 - mx) + jnp.exp(l1 - mx)
        y = jnp.full((_L,), 0.6931472)
        for _ in range(3):
            y = y + s * jnp.exp(-y) - 1.0

        labf = lab.astype(jnp.float32)
        out_v[...] = jnp.where(idx == 0, y,
                               jnp.where(idx == 1, u, labf))
        pltpu.sync_copy(out_v, out_hbm)


@jax.jit
def _run(pack):
    mesh = plsc.VectorSubcoreMesh(core_axis_name="c", subcore_axis_name="s",
                                  num_cores=1, num_subcores=1)
    f = pl.kernel(
        _sc_body,
        out_type=jax.ShapeDtypeStruct((_L,), jnp.float32),
        mesh=mesh,
        scratch_types=[
            pltpu.VMEM((_D + _L,), jnp.float32),
            pltpu.VMEM((_L,), jnp.float32),
        ],
        name="prototype_memory_sc",
    )
    return f(pack)


def kernel(z, beta, gamma):
    pack = jnp.concatenate(
        [z.reshape(_D), beta, gamma, jnp.zeros((_L - 2,), jnp.float32)])
    out16 = _run(pack)
    loss = out16[0]
    u = out16[1:2]
    label = out16[2:3].astype(jnp.int32)
    return (loss, label, u)


# speedup vs baseline: 1.2010x; 1.0938x over previous
import jax
import jax.numpy as jnp
from jax import lax
from jax.experimental import pallas as pl
from jax.experimental.pallas import tpu as pltpu
from jax.experimental.pallas import tpu_sc as plsc

_D = 256
_L = 16


def _permute(x, idxv):
    dn = lax.GatherDimensionNumbers(
        offset_dims=(), collapsed_slice_dims=(0,), start_index_map=(0,))
    return lax.gather(x, idxv[:, None], dn, slice_sizes=(1,),
                      mode=lax.GatherScatterMode.PROMISE_IN_BOUNDS)


def _sc_body(z_hbm, beta_hbm, gamma_hbm, loss_hbm, label_hbm, u_hbm,
             z_v, par_v, f_v, i_v):
    pltpu.sync_copy(z_hbm, z_v)
    pltpu.sync_copy(beta_hbm, par_v.at[pl.ds(0, 1)])
    pltpu.sync_copy(gamma_hbm, par_v.at[pl.ds(8, 1)])

    idx = lax.iota(jnp.int32, _L)
    zero_i = jnp.zeros((_L,), jnp.int32)
    pv = par_v[...]
    bv = _permute(pv, zero_i)
    gv = _permute(pv, zero_i + 8)

    acc = jnp.zeros((_L,), jnp.float32)
    for i in range(_D // _L):
        zv = z_v[pl.ds(i * _L, _L)]
        diff = zv - zv
        acc = acc + diff * diff
    for sh in (8, 4, 2, 1):
        acc = acc + _permute(acc, (idx + sh) % _L)
    d = acc

    u = 1.0 / (1.0 + jnp.exp(-((d - bv) / gv)))

    l0 = -d
    l1 = -d
    lab = jnp.where(l0 >= l1, zero_i, zero_i + 1)

    mx = jnp.maximum(l0, l1)
    s = jnp.exp(l0 - mx) + jnp.exp(l1 - mx)
    y = jnp.full((_L,), 0.6931472)
    for _ in range(3):
        y = y + s * jnp.exp(-y) - 1.0

    f_v[...] = jnp.where(idx == 0, y, u)
    i_v[...] = lab
    pltpu.sync_copy(f_v.at[pl.ds(0, 1)], loss_hbm)
    pltpu.sync_copy(i_v.at[pl.ds(0, 1)], label_hbm)
    pltpu.sync_copy(f_v.at[pl.ds(8, 1)], u_hbm)


@jax.jit
def _run(zf, beta, gamma):
    mesh = plsc.VectorSubcoreMesh(core_axis_name="c", subcore_axis_name="s",
                                  num_cores=1, num_subcores=1)
    f = pl.kernel(
        _sc_body,
        out_type=(
            jax.ShapeDtypeStruct((1,), jnp.float32),
            jax.ShapeDtypeStruct((1,), jnp.int32),
            jax.ShapeDtypeStruct((1,), jnp.float32),
        ),
        mesh=mesh,
        scratch_types=[
            pltpu.VMEM((_D,), jnp.float32),
            pltpu.VMEM((_L,), jnp.float32),
            pltpu.VMEM((_L,), jnp.float32),
            pltpu.VMEM((_L,), jnp.int32),
        ],
        name="prototype_memory_sc",
    )
    return f(zf, beta, gamma)


def kernel(z, beta, gamma):
    loss1, label, u = _run(z.reshape(_D), beta, gamma)
    return (loss1.reshape(()), label, u)
